# Initial kernel scaffold; baseline (speedup 1.0000x reference)
import jax
import jax.numpy as jnp
from jax.experimental import pallas as pl


def kernel(x, edge_index, edge_attr, W1, W2, ln_gamma, ln_beta):
    raise NotImplementedError("write your pallas kernel here")


# speedup vs baseline: 3.9248x; 3.9248x over previous
import functools

import jax
import jax.numpy as jnp
from jax import lax
from jax.experimental import pallas as pl
from jax.experimental.pallas import tpu as pltpu
from jax.experimental.pallas import tpu_sc as plsc

NC = 2
NS = 16
LANES = 16


def _edge_stage(x, src, dst, edge_attr):
    N, D = x.shape
    E = src.shape[0]
    NW = NC * NS
    C = 80
    assert E % (NW * C) == 0 and N % NS == 0 and D % LANES == 0
    EPW = E // NW
    NCHUNK = EPW // C
    ZR = 80
    assert N % ZR == 0
    NZ = N // ZR
    NZI = -(-NZ // NS)
    KD = D // LANES

    mesh = plsc.VectorSubcoreMesh(
        core_axis_name="c", subcore_axis_name="s", num_cores=NC, num_subcores=NS
    )

    @functools.partial(
        pl.kernel,
        out_type=jax.ShapeDtypeStruct((NC * N, D), jnp.float32),
        mesh=mesh,
        scratch_types=[
            pltpu.VMEM((C,), jnp.int32),
            pltpu.VMEM((C,), jnp.int32),
            pltpu.VMEM((C, D), jnp.float32),
            pltpu.VMEM((C, D), jnp.float32),
            pltpu.VMEM((ZR, D), jnp.float32),
            pltpu.VMEM_SHARED((N, D), jnp.float32),
            pltpu.SemaphoreType.DMA,
            pltpu.SemaphoreType.DMA,
        ],
    )
    def edge_kernel(x_hbm, src_hbm, dst_hbm, ea_hbm, out_hbm,
                    sidx, didx, xr, ea, zbuf, agg_sh, sem_g, sem_e):
        c = lax.axis_index("c")
        s = lax.axis_index("s")
        wid = s * NC + c

        def zero_row(r, _):
            for k in range(KD):
                zbuf[r, pl.ds(k * LANES, LANES)] = jnp.zeros((LANES,), jnp.float32)
            return 0
        lax.fori_loop(0, ZR, zero_row, 0)
        for i in range(NZI):
            q = s + i * NS

            @pl.when(q < NZ)
            def _():
                pltpu.sync_copy(zbuf, agg_sh.at[pl.ds(q * ZR, ZR)])
        plsc.subcore_barrier()

        def chunk(t, _):
            ebase = wid * EPW + t * C
            pltpu.sync_copy(src_hbm.at[pl.ds(ebase, C)], sidx)
            pltpu.sync_copy(dst_hbm.at[pl.ds(ebase, C)], didx)
            g = pltpu.async_copy(x_hbm.at[sidx], xr, sem_g)
            e = pltpu.async_copy(ea_hbm.at[pl.ds(ebase, C)], ea, sem_e)
            g.wait()
            e.wait()

            def msg_row(r, _):
                for k in range(KD):
                    v = xr[r, pl.ds(k * LANES, LANES)] + ea[r, pl.ds(k * LANES, LANES)]
                    ea[r, pl.ds(k * LANES, LANES)] = jnp.maximum(v, 0.0)
                return 0
            lax.fori_loop(0, C, msg_row, 0)

            pltpu.sync_copy(ea, agg_sh.at[didx], add=True)
            return 0
        lax.fori_loop(0, NCHUNK, chunk, 0)
        plsc.subcore_barrier()

        for i in range(NZI):
            q = s + i * NS

            @pl.when(q < NZ)
            def _():
                pltpu.sync_copy(agg_sh.at[pl.ds(q * ZR, ZR)],
                                out_hbm.at[pl.ds(c * N + q * ZR, ZR)])

    return edge_kernel(x, src, dst, edge_attr)


def _dense_stage(x, agg2, W1, W2, ln_gamma, ln_beta):
    N, D = x.shape
    H = W1.shape[1]
    BN = 200
    assert N % BN == 0

    def body(x_ref, a0_ref, a1_ref, w1_ref, w2_ref, g_ref, b_ref, o_ref):
        xb = x_ref[...]
        h = xb + a0_ref[...] + a1_ref[...]
        t = jnp.dot(h, w1_ref[...], preferred_element_type=jnp.float32)
        t = 0.5 * t * (1.0 + lax.erf(t * 0.7071067811865476))
        y = xb + jnp.dot(t, w2_ref[...], preferred_element_type=jnp.float32)
        mean = jnp.mean(y, axis=1, keepdims=True)
        var = jnp.mean((y - mean) ** 2, axis=1, keepdims=True)
        o_ref[...] = (y - mean) * lax.rsqrt(var + 1e-5) * g_ref[...] + b_ref[...]

    nb = N // BN
    return pl.pallas_call(
        body,
        grid=(nb,),
        in_specs=[
            pl.BlockSpec((BN, D), lambda i: (i, 0)),
            pl.BlockSpec((BN, D), lambda i: (i, 0)),
            pl.BlockSpec((BN, D), lambda i: (i + nb, 0)),
            pl.BlockSpec((D, H), lambda i: (0, 0)),
            pl.BlockSpec((H, D), lambda i: (0, 0)),
            pl.BlockSpec((1, D), lambda i: (0, 0)),
            pl.BlockSpec((1, D), lambda i: (0, 0)),
        ],
        out_specs=pl.BlockSpec((BN, D), lambda i: (i, 0)),
        out_shape=jax.ShapeDtypeStruct((N, D), jnp.float32),
    )(x, agg2, agg2, W1, W2, ln_gamma.reshape(1, D), ln_beta.reshape(1, D))


def kernel(x, edge_index, edge_attr, W1, W2, ln_gamma, ln_beta):
    src = edge_index[0]
    dst = edge_index[1]
    agg2 = _edge_stage(x, src, dst, edge_attr)
    return _dense_stage(x, agg2, W1, W2, ln_gamma, ln_beta)


# speedup vs baseline: 6.7576x; 1.7218x over previous
import functools

import jax
import jax.numpy as jnp
from jax import lax
from jax.experimental import pallas as pl
from jax.experimental.pallas import tpu as pltpu
from jax.experimental.pallas import tpu_sc as plsc

NC = 2
NS = 16
LANES = 16


def _edge_stage(x, src, dst, edge_attr):
    N, D = x.shape
    E = src.shape[0]
    NW = NC * NS
    C = 40
    NH = 5
    assert E % (NW * C * NH) == 0 and N % NS == 0 and D % LANES == 0
    EPW = E // NW
    NCHUNK = EPW // C
    NCH = NCHUNK // NH
    assert NCH % 2 == 0 and NCH >= 4
    ZR = 40
    WR = 400
    assert N % ZR == 0 and N % WR == 0
    NZ = N // ZR
    NZI = -(-NZ // NS)
    NWO = N // WR
    NWI = -(-NWO // NS)
    KD = D // LANES

    src4 = src.reshape(NW, NH, NCH, C)
    dst4 = dst.reshape(NW, NH, NCH, C)

    mesh = plsc.VectorSubcoreMesh(
        core_axis_name="c", subcore_axis_name="s", num_cores=NC, num_subcores=NS
    )

    @functools.partial(
        pl.kernel,
        out_type=jax.ShapeDtypeStruct((NC * N, D), jnp.float32),
        mesh=mesh,
        scratch_types=[
            pltpu.VMEM((NCH, C), jnp.int32),
            pltpu.VMEM((NCH, C), jnp.int32),
            pltpu.VMEM((2, C, D), jnp.float32),
            pltpu.VMEM((2, C, D), jnp.float32),
            pltpu.VMEM_SHARED((N, D), jnp.float32),
            pltpu.SemaphoreType.DMA,
            pltpu.SemaphoreType.DMA, pltpu.SemaphoreType.DMA,
            pltpu.SemaphoreType.DMA, pltpu.SemaphoreType.DMA,
        ],
    )
    def edge_kernel(x_hbm, src_hbm, dst_hbm, ea_hbm, out_hbm,
                    sidx2, didx2, xr, ea, agg_sh,
                    sem_i, sem_g0, sem_g1, sem_e0, sem_e1):
        c = lax.axis_index("c")
        s = lax.axis_index("s")
        wid = s * NC + c
        sems_g = (sem_g0, sem_g1)
        sems_e = (sem_e0, sem_e1)

        i1 = pltpu.async_copy(src_hbm.at[wid, 0], sidx2, sem_i)
        i2 = pltpu.async_copy(dst_hbm.at[wid, 0], didx2, sem_i)

        def zero_row(r, _):
            for k in range(KD):
                xr[0, r, pl.ds(k * LANES, LANES)] = jnp.zeros((LANES,), jnp.float32)
            return 0
        lax.fori_loop(0, ZR, zero_row, 0)
        for i in range(NZI):
            q = s + i * NS

            @pl.when(q < NZ)
            def _():
                pltpu.sync_copy(xr.at[0], agg_sh.at[pl.ds(q * ZR, ZR)])
        i1.wait()
        i2.wait()
        plsc.subcore_barrier()

        def gather_copy(t, b):
            return pltpu.make_async_copy(x_hbm.at[sidx2.at[t]], xr.at[b], sems_g[b])

        def ea_copy(h, t, b):
            base = wid * EPW + (h * NCH + t) * C
            return pltpu.make_async_copy(
                ea_hbm.at[pl.ds(base, C)], ea.at[b], sems_e[b])

        def compute(b):
            xrb, eab = xr.at[b], ea.at[b]

            @plsc.parallel_loop(0, C, 1, unroll=4)
            def _(r):
                for k in range(KD):
                    sl = pl.ds(k * LANES, LANES)
                    eab[r, sl] = jnp.maximum(xrb[r, sl] + eab[r, sl], 0.0)

        for h in range(NH):
            if h > 0:
                pltpu.sync_copy(src_hbm.at[wid, h], sidx2)
                pltpu.sync_copy(dst_hbm.at[wid, h], didx2)

            def step(t, b, issue_next_ea, h=h):
                gather_copy(t, b).wait()
                ea_copy(h, t, b).wait()
                compute(b)
                sd = pltpu.async_copy(ea.at[b], agg_sh.at[didx2.at[t]], sems_e[b],
                                      add=True)
                @pl.when(issue_next_ea)
                def _():
                    gather_copy(t + 2, b).start()
                sd.wait()

                @pl.when(issue_next_ea)
                def _():
                    ea_copy(h, t + 2, b).start()

            gather_copy(0, 0).start()
            ea_copy(h, 0, 0).start()
            gather_copy(1, 1).start()
            ea_copy(h, 1, 1).start()

            def pair(u, _):
                step(2 * u, 0, 2 * u + 2 < NCH)
                step(2 * u + 1, 1, 2 * u + 3 < NCH)
                return 0
            lax.fori_loop(0, NCH // 2, pair, 0)
        plsc.subcore_barrier()

        for i in range(NWI):
            q = s + i * NS

            @pl.when(q < NWO)
            def _():
                pltpu.sync_copy(agg_sh.at[pl.ds(q * WR, WR)],
                                out_hbm.at[pl.ds(c * N + q * WR, WR)])

    return edge_kernel(x, src4, dst4, edge_attr)


def _dense_stage(x, agg2, W1, W2, ln_gamma, ln_beta):
    N, D = x.shape
    H = W1.shape[1]
    BN = 200
    assert N % BN == 0

    def body(x_ref, a0_ref, a1_ref, w1_ref, w2_ref, g_ref, b_ref, o_ref):
        xb = x_ref[...]
        h = xb + a0_ref[...] + a1_ref[...]
        t = jnp.dot(h, w1_ref[...], preferred_element_type=jnp.float32)
        t = 0.5 * t * (1.0 + lax.erf(t * 0.7071067811865476))
        y = xb + jnp.dot(t, w2_ref[...], preferred_element_type=jnp.float32)
        mean = jnp.mean(y, axis=1, keepdims=True)
        var = jnp.mean((y - mean) ** 2, axis=1, keepdims=True)
        o_ref[...] = (y - mean) * lax.rsqrt(var + 1e-5) * g_ref[...] + b_ref[...]

    nb = N // BN
    return pl.pallas_call(
        body,
        grid=(nb,),
        in_specs=[
            pl.BlockSpec((BN, D), lambda i: (i, 0)),
            pl.BlockSpec((BN, D), lambda i: (i, 0)),
            pl.BlockSpec((BN, D), lambda i: (i + nb, 0)),
            pl.BlockSpec((D, H), lambda i: (0, 0)),
            pl.BlockSpec((H, D), lambda i: (0, 0)),
            pl.BlockSpec((1, D), lambda i: (0, 0)),
            pl.BlockSpec((1, D), lambda i: (0, 0)),
        ],
        out_specs=pl.BlockSpec((BN, D), lambda i: (i, 0)),
        out_shape=jax.ShapeDtypeStruct((N, D), jnp.float32),
    )(x, agg2, agg2, W1, W2, ln_gamma.reshape(1, D), ln_beta.reshape(1, D))


def kernel(x, edge_index, edge_attr, W1, W2, ln_gamma, ln_beta):
    src = edge_index[0]
    dst = edge_index[1]
    agg2 = _edge_stage(x, src, dst, edge_attr)
    return _dense_stage(x, agg2, W1, W2, ln_gamma, ln_beta)


# speedup vs baseline: 7.3755x; 1.0914x over previous
import functools

import jax
import jax.numpy as jnp
from jax import lax
from jax.experimental import pallas as pl
from jax.experimental.pallas import tpu as pltpu
from jax.experimental.pallas import tpu_sc as plsc

NC = 2
NS = 16
LANES = 16


def _edge_stage(x, src, dst, edge_attr):
    N, D = x.shape
    E = src.shape[0]
    NW = NC * NS
    C = 40
    NH = 5
    assert E % (NW * C * NH) == 0 and N % NS == 0 and D % LANES == 0
    EPW = E // NW
    NCHUNK = EPW // C
    NCH = NCHUNK // NH
    assert NCH % 2 == 0 and NCH >= 4
    ZR = 40
    WR = 400
    assert N % ZR == 0 and N % WR == 0
    NZ = N // ZR
    NZI = -(-NZ // NS)
    NWO = N // WR
    NWI = -(-NWO // NS)
    KD = D // LANES

    src4 = src.reshape(NW, NH, NCH, C)
    dst4 = dst.reshape(NW, NH, NCH, C)

    mesh = plsc.VectorSubcoreMesh(
        core_axis_name="c", subcore_axis_name="s", num_cores=NC, num_subcores=NS
    )

    @functools.partial(
        pl.kernel,
        out_type=jax.ShapeDtypeStruct((NC * N, D), jnp.float32),
        mesh=mesh,
        scratch_types=[
            pltpu.VMEM((NCH, C), jnp.int32),
            pltpu.VMEM((NCH, C), jnp.int32),
            pltpu.VMEM((2, C, D), jnp.float32),
            pltpu.VMEM((2, C, D), jnp.float32),
            pltpu.VMEM((C, D), jnp.float32),
            pltpu.VMEM_SHARED((N, D), jnp.float32),
            pltpu.SemaphoreType.DMA,
            pltpu.SemaphoreType.DMA, pltpu.SemaphoreType.DMA,
            pltpu.SemaphoreType.DMA, pltpu.SemaphoreType.DMA,
            pltpu.SemaphoreType.DMA,
        ],
    )
    def edge_kernel(x_hbm, src_hbm, dst_hbm, ea_hbm, out_hbm,
                    sidx2, didx2, xr, ea, msg, agg_sh,
                    sem_i, sem_g0, sem_g1, sem_e0, sem_e1, sem_s):
        c = lax.axis_index("c")
        s = lax.axis_index("s")
        wid = s * NC + c
        sems_g = (sem_g0, sem_g1)
        sems_e = (sem_e0, sem_e1)

        i1 = pltpu.async_copy(src_hbm.at[wid, 0], sidx2, sem_i)
        i2 = pltpu.async_copy(dst_hbm.at[wid, 0], didx2, sem_i)

        def zero_row(r, _):
            for k in range(KD):
                xr[0, r, pl.ds(k * LANES, LANES)] = jnp.zeros((LANES,), jnp.float32)
            return 0
        lax.fori_loop(0, ZR, zero_row, 0)
        for i in range(NZI):
            q = s + i * NS

            @pl.when(q < NZ)
            def _():
                pltpu.sync_copy(xr.at[0], agg_sh.at[pl.ds(q * ZR, ZR)])
        i1.wait()
        i2.wait()
        plsc.subcore_barrier()

        def gather_copy(t, b):
            return pltpu.make_async_copy(x_hbm.at[sidx2.at[t]], xr.at[b], sems_g[b])

        def ea_copy(h, t, b):
            base = wid * EPW + (h * NCH + t) * C
            return pltpu.make_async_copy(
                ea_hbm.at[pl.ds(base, C)], ea.at[b], sems_e[b])

        def compute(b):
            xrb, eab = xr.at[b], ea.at[b]

            @plsc.parallel_loop(0, C, 1, unroll=4)
            def _(r):
                for k in range(KD):
                    sl = pl.ds(k * LANES, LANES)
                    msg[r, sl] = jnp.maximum(xrb[r, sl] + eab[r, sl], 0.0)

        for h in range(NH):
            if h > 0:
                pltpu.sync_copy(src_hbm.at[wid, h], sidx2)
                pltpu.sync_copy(dst_hbm.at[wid, h], didx2)

            def step(t, b, issue_next, h=h):
                gather_copy(t, b).wait()
                ea_copy(h, t, b).wait()
                compute(b)
                sd = pltpu.async_copy(msg, agg_sh.at[didx2.at[t]], sem_s, add=True)
                @pl.when(issue_next)
                def _():
                    gather_copy(t + 2, b).start()
                    ea_copy(h, t + 2, b).start()
                sd.wait()

            gather_copy(0, 0).start()
            ea_copy(h, 0, 0).start()
            gather_copy(1, 1).start()
            ea_copy(h, 1, 1).start()

            def pair(u, _):
                step(2 * u, 0, 2 * u + 2 < NCH)
                step(2 * u + 1, 1, 2 * u + 3 < NCH)
                return 0
            lax.fori_loop(0, NCH // 2, pair, 0)
        plsc.subcore_barrier()

        for i in range(NWI):
            q = s + i * NS

            @pl.when(q < NWO)
            def _():
                pltpu.sync_copy(agg_sh.at[pl.ds(q * WR, WR)],
                                out_hbm.at[pl.ds(c * N + q * WR, WR)])

    return edge_kernel(x, src4, dst4, edge_attr)


def _dense_stage(x, agg2, W1, W2, ln_gamma, ln_beta):
    N, D = x.shape
    H = W1.shape[1]
    BN = 400
    assert N % BN == 0

    def body(x_ref, a0_ref, a1_ref, w1_ref, w2_ref, g_ref, b_ref, o_ref):
        xb = x_ref[...]
        h = xb + a0_ref[...] + a1_ref[...]
        t = jnp.dot(h, w1_ref[...], preferred_element_type=jnp.float32)
        t = 0.5 * t * (1.0 + lax.erf(t * 0.7071067811865476))
        y = xb + jnp.dot(t, w2_ref[...], preferred_element_type=jnp.float32)
        mean = jnp.mean(y, axis=1, keepdims=True)
        var = jnp.mean((y - mean) ** 2, axis=1, keepdims=True)
        o_ref[...] = (y - mean) * lax.rsqrt(var + 1e-5) * g_ref[...] + b_ref[...]

    nb = N // BN
    return pl.pallas_call(
        body,
        grid=(nb,),
        in_specs=[
            pl.BlockSpec((BN, D), lambda i: (i, 0)),
            pl.BlockSpec((BN, D), lambda i: (i, 0)),
            pl.BlockSpec((BN, D), lambda i: (i + nb, 0)),
            pl.BlockSpec((D, H), lambda i: (0, 0)),
            pl.BlockSpec((H, D), lambda i: (0, 0)),
            pl.BlockSpec((1, D), lambda i: (0, 0)),
            pl.BlockSpec((1, D), lambda i: (0, 0)),
        ],
        out_specs=pl.BlockSpec((BN, D), lambda i: (i, 0)),
        out_shape=jax.ShapeDtypeStruct((N, D), jnp.float32),
    )(x, agg2, agg2, W1, W2, ln_gamma.reshape(1, D), ln_beta.reshape(1, D))


def kernel(x, edge_index, edge_attr, W1, W2, ln_gamma, ln_beta):
    src = edge_index[0]
    dst = edge_index[1]
    agg2 = _edge_stage(x, src, dst, edge_attr)
    return _dense_stage(x, agg2, W1, W2, ln_gamma, ln_beta)


# speedup vs baseline: 7.3762x; 1.0001x over previous
import functools

import jax
import jax.numpy as jnp
from jax import lax
from jax.experimental import pallas as pl
from jax.experimental.pallas import tpu as pltpu
from jax.experimental.pallas import tpu_sc as plsc

NC = 2
NS = 16
LANES = 16


def _edge_stage(x, src, dst, edge_attr):
    N, D = x.shape
    E = src.shape[0]
    NW = NC * NS
    C = 40
    NH = 5
    assert E % (NW * C * NH) == 0 and N % NS == 0 and D % LANES == 0
    EPW = E // NW
    NCHUNK = EPW // C
    NCH = NCHUNK // NH
    assert NCH % 2 == 0 and NCH >= 4
    ZR = 40
    WR = 400
    assert N % ZR == 0 and N % WR == 0
    NZ = N // ZR
    NZI = -(-NZ // NS)
    NWO = N // WR
    NWI = -(-NWO // NS)
    KD = D // LANES

    src4 = src.reshape(NW, NH, NCH, C)
    dst4 = dst.reshape(NW, NH, NCH, C)

    mesh = plsc.VectorSubcoreMesh(
        core_axis_name="c", subcore_axis_name="s", num_cores=NC, num_subcores=NS
    )

    @functools.partial(
        pl.kernel,
        out_type=jax.ShapeDtypeStruct((NC * N, D), jnp.float32),
        mesh=mesh,
        scratch_types=[
            pltpu.VMEM((NCH, C), jnp.int32),
            pltpu.VMEM((NCH, C), jnp.int32),
            pltpu.VMEM((2, C, D), jnp.float32),
            pltpu.VMEM((2, C, D), jnp.float32),
            pltpu.VMEM((C, D), jnp.float32),
            pltpu.VMEM_SHARED((N, D), jnp.float32),
            pltpu.SemaphoreType.DMA,
            pltpu.SemaphoreType.DMA, pltpu.SemaphoreType.DMA,
            pltpu.SemaphoreType.DMA, pltpu.SemaphoreType.DMA,
            pltpu.SemaphoreType.DMA,
        ],
    )
    def edge_kernel(x_hbm, src_hbm, dst_hbm, ea_hbm, out_hbm,
                    sidx2, didx2, xr, ea, msg, agg_sh,
                    sem_i, sem_g0, sem_g1, sem_e0, sem_e1, sem_s):
        c = lax.axis_index("c")
        s = lax.axis_index("s")
        wid = s * NC + c
        sems_g = (sem_g0, sem_g1)
        sems_e = (sem_e0, sem_e1)

        i1 = pltpu.async_copy(src_hbm.at[wid, 0], sidx2, sem_i)
        i2 = pltpu.async_copy(dst_hbm.at[wid, 0], didx2, sem_i)

        def zero_row(r, _):
            for k in range(KD):
                xr[0, r, pl.ds(k * LANES, LANES)] = jnp.zeros((LANES,), jnp.float32)
            return 0
        lax.fori_loop(0, ZR, zero_row, 0)
        for i in range(NZI):
            q = s + i * NS

            @pl.when(q < NZ)
            def _():
                pltpu.sync_copy(xr.at[0], agg_sh.at[pl.ds(q * ZR, ZR)])
        i1.wait()
        i2.wait()
        plsc.subcore_barrier()

        def gather_copy(t, b):
            return pltpu.make_async_copy(x_hbm.at[sidx2.at[t]], xr.at[b], sems_g[b])

        def ea_copy(h, t, b):
            base = wid * EPW + (h * NCH + t) * C
            return pltpu.make_async_copy(
                ea_hbm.at[pl.ds(base, C)], ea.at[b], sems_e[b])

        def compute(b):
            xrb, eab = xr.at[b], ea.at[b]

            @plsc.parallel_loop(0, C, 1, unroll=4)
            def _(r):
                for k in range(KD):
                    sl = pl.ds(k * LANES, LANES)
                    msg[r, sl] = jnp.maximum(xrb[r, sl] + eab[r, sl], 0.0)

        for h in range(NH):
            if h > 0:
                pltpu.sync_copy(src_hbm.at[wid, h], sidx2)
                pltpu.sync_copy(dst_hbm.at[wid, h], didx2)

            def step(t, b, issue_next, h=h):
                gather_copy(t, b).wait()
                ea_copy(h, t, b).wait()

                @pl.when(t >= 1)
                def _():
                    pltpu.make_async_copy(
                        msg, agg_sh.at[didx2.at[t - 1]], sem_s).wait()
                compute(b)
                pltpu.async_copy(msg, agg_sh.at[didx2.at[t]], sem_s, add=True)

                @pl.when(issue_next)
                def _():
                    gather_copy(t + 2, b).start()
                    ea_copy(h, t + 2, b).start()

            gather_copy(0, 0).start()
            ea_copy(h, 0, 0).start()
            gather_copy(1, 1).start()
            ea_copy(h, 1, 1).start()

            def pair(u, _):
                step(2 * u, 0, 2 * u + 2 < NCH)
                step(2 * u + 1, 1, 2 * u + 3 < NCH)
                return 0
            lax.fori_loop(0, NCH // 2, pair, 0)
            pltpu.make_async_copy(
                msg, agg_sh.at[didx2.at[NCH - 1]], sem_s).wait()
        plsc.subcore_barrier()

        for i in range(NWI):
            q = s + i * NS

            @pl.when(q < NWO)
            def _():
                pltpu.sync_copy(agg_sh.at[pl.ds(q * WR, WR)],
                                out_hbm.at[pl.ds(c * N + q * WR, WR)])

    return edge_kernel(x, src4, dst4, edge_attr)


def _dense_stage(x, agg2, W1, W2, ln_gamma, ln_beta):
    N, D = x.shape
    H = W1.shape[1]
    BN = 400
    assert N % BN == 0

    def body(x_ref, a0_ref, a1_ref, w1_ref, w2_ref, g_ref, b_ref, o_ref):
        xb = x_ref[...]
        h = xb + a0_ref[...] + a1_ref[...]
        t = jnp.dot(h, w1_ref[...], preferred_element_type=jnp.float32)
        t = 0.5 * t * (1.0 + lax.erf(t * 0.7071067811865476))
        y = xb + jnp.dot(t, w2_ref[...], preferred_element_type=jnp.float32)
        mean = jnp.mean(y, axis=1, keepdims=True)
        var = jnp.mean((y - mean) ** 2, axis=1, keepdims=True)
        o_ref[...] = (y - mean) * lax.rsqrt(var + 1e-5) * g_ref[...] + b_ref[...]

    nb = N // BN
    return pl.pallas_call(
        body,
        grid=(nb,),
        in_specs=[
            pl.BlockSpec((BN, D), lambda i: (i, 0)),
            pl.BlockSpec((BN, D), lambda i: (i, 0)),
            pl.BlockSpec((BN, D), lambda i: (i + nb, 0)),
            pl.BlockSpec((D, H), lambda i: (0, 0)),
            pl.BlockSpec((H, D), lambda i: (0, 0)),
            pl.BlockSpec((1, D), lambda i: (0, 0)),
            pl.BlockSpec((1, D), lambda i: (0, 0)),
        ],
        out_specs=pl.BlockSpec((BN, D), lambda i: (i, 0)),
        out_shape=jax.ShapeDtypeStruct((N, D), jnp.float32),
    )(x, agg2, agg2, W1, W2, ln_gamma.reshape(1, D), ln_beta.reshape(1, D))


def kernel(x, edge_index, edge_attr, W1, W2, ln_gamma, ln_beta):
    src = edge_index[0]
    dst = edge_index[1]
    agg2 = _edge_stage(x, src, dst, edge_attr)
    return _dense_stage(x, agg2, W1, W2, ln_gamma, ln_beta)


# speedup vs baseline: 7.4915x; 1.0156x over previous
import functools

import jax
import jax.numpy as jnp
from jax import lax
from jax.experimental import pallas as pl
from jax.experimental.pallas import tpu as pltpu
from jax.experimental.pallas import tpu_sc as plsc

NC = 2
NS = 16
LANES = 16


def _edge_stage(x, src, dst, edge_attr):
    N, D = x.shape
    E = src.shape[0]
    NW = NC * NS
    C = 40
    NH = 5
    assert E % (NW * C * NH) == 0 and N % NS == 0 and D % LANES == 0
    EPW = E // NW
    NCHUNK = EPW // C
    NCH = NCHUNK // NH
    assert NCH % 2 == 0 and NCH >= 4
    ZR = 40
    WR = 400
    assert N % ZR == 0 and N % WR == 0
    NZ = N // ZR
    NZI = -(-NZ // NS)
    NWO = N // WR
    NWI = -(-NWO // NS)
    KD = D // LANES

    src4 = src.reshape(NW, NH, NCH, C)
    dst4 = dst.reshape(NW, NH, NCH, C)

    DW = D // 2
    u = jax.lax.bitcast_convert_type(
        x.astype(jnp.bfloat16), jnp.uint16).astype(jnp.uint32)
    j = jnp.arange(DW)
    cols_lo = 32 * (j // LANES) + j % LANES
    xp = jax.lax.bitcast_convert_type(
        u[:, cols_lo] | (u[:, cols_lo + LANES] << 16), jnp.int32)

    mesh = plsc.VectorSubcoreMesh(
        core_axis_name="c", subcore_axis_name="s", num_cores=NC, num_subcores=NS
    )

    @functools.partial(
        pl.kernel,
        out_type=jax.ShapeDtypeStruct((NC * N, D), jnp.float32),
        mesh=mesh,
        compiler_params=pltpu.CompilerParams(use_tc_tiling_on_sc=False),
        scratch_types=[
            pltpu.VMEM((NCH, C), jnp.int32),
            pltpu.VMEM((NCH, C), jnp.int32),
            pltpu.VMEM((2, C, DW), jnp.int32),
            pltpu.VMEM((2, C, D), jnp.float32),
            pltpu.VMEM((C, D), jnp.float32),
            pltpu.VMEM_SHARED((N, D), jnp.float32),
            pltpu.SemaphoreType.DMA,
            pltpu.SemaphoreType.DMA, pltpu.SemaphoreType.DMA,
            pltpu.SemaphoreType.DMA, pltpu.SemaphoreType.DMA,
            pltpu.SemaphoreType.DMA,
        ],
    )
    def edge_kernel(x_hbm, src_hbm, dst_hbm, ea_hbm, out_hbm,
                    sidx2, didx2, xr, ea, msg, agg_sh,
                    sem_i, sem_g0, sem_g1, sem_e0, sem_e1, sem_s):
        c = lax.axis_index("c")
        s = lax.axis_index("s")
        wid = s * NC + c
        sems_g = (sem_g0, sem_g1)
        sems_e = (sem_e0, sem_e1)

        i1 = pltpu.async_copy(src_hbm.at[wid, 0], sidx2, sem_i)
        i2 = pltpu.async_copy(dst_hbm.at[wid, 0], didx2, sem_i)

        def zero_row(r, _):
            for k in range(KD):
                msg[r, pl.ds(k * LANES, LANES)] = jnp.zeros((LANES,), jnp.float32)
            return 0
        lax.fori_loop(0, ZR, zero_row, 0)
        for i in range(NZI):
            q = s + i * NS

            @pl.when(q < NZ)
            def _():
                pltpu.sync_copy(msg, agg_sh.at[pl.ds(q * ZR, ZR)])
        i1.wait()
        i2.wait()
        plsc.subcore_barrier()

        def gather_copy(t, b):
            return pltpu.make_async_copy(x_hbm.at[sidx2.at[t]], xr.at[b], sems_g[b])

        def ea_copy(h, t, b):
            base = wid * EPW + (h * NCH + t) * C
            return pltpu.make_async_copy(
                ea_hbm.at[pl.ds(base, C)], ea.at[b], sems_e[b])

        def compute(b):
            xrb, eab = xr.at[b], ea.at[b]
            himask = jnp.int32(-65536)

            @plsc.parallel_loop(0, C, 1, unroll=4)
            def _(r):
                for k in range(KD // 2):
                    w = xrb[r, pl.ds(k * LANES, LANES)]
                    lo = lax.bitcast_convert_type(w << 16, jnp.float32)
                    hi = lax.bitcast_convert_type(w & himask, jnp.float32)
                    sl_lo = pl.ds(2 * k * LANES, LANES)
                    sl_hi = pl.ds((2 * k + 1) * LANES, LANES)
                    msg[r, sl_lo] = jnp.maximum(lo + eab[r, sl_lo], 0.0)
                    msg[r, sl_hi] = jnp.maximum(hi + eab[r, sl_hi], 0.0)

        for h in range(NH):
            if h > 0:
                pltpu.sync_copy(src_hbm.at[wid, h], sidx2)
                pltpu.sync_copy(dst_hbm.at[wid, h], didx2)

            def step(t, b, issue_next, h=h):
                gather_copy(t, b).wait()
                ea_copy(h, t, b).wait()

                @pl.when(t >= 1)
                def _():
                    pltpu.make_async_copy(
                        msg, agg_sh.at[didx2.at[t - 1]], sem_s).wait()
                compute(b)
                pltpu.async_copy(msg, agg_sh.at[didx2.at[t]], sem_s, add=True)

                @pl.when(issue_next)
                def _():
                    gather_copy(t + 2, b).start()
                    ea_copy(h, t + 2, b).start()

            gather_copy(0, 0).start()
            ea_copy(h, 0, 0).start()
            gather_copy(1, 1).start()
            ea_copy(h, 1, 1).start()

            def pair(u, _):
                step(2 * u, 0, 2 * u + 2 < NCH)
                step(2 * u + 1, 1, 2 * u + 3 < NCH)
                return 0
            lax.fori_loop(0, NCH // 2, pair, 0)
            pltpu.make_async_copy(
                msg, agg_sh.at[didx2.at[NCH - 1]], sem_s).wait()
        plsc.subcore_barrier()

        for i in range(NWI):
            q = s + i * NS

            @pl.when(q < NWO)
            def _():
                pltpu.sync_copy(agg_sh.at[pl.ds(q * WR, WR)],
                                out_hbm.at[pl.ds(c * N + q * WR, WR)])

    return edge_kernel(xp, src4, dst4, edge_attr)


def _dense_stage(x, agg2, W1, W2, ln_gamma, ln_beta):
    N, D = x.shape
    H = W1.shape[1]
    BN = 400
    assert N % BN == 0

    def body(x_ref, a0_ref, a1_ref, w1_ref, w2_ref, g_ref, b_ref, o_ref):
        xb = x_ref[...]
        h = xb + a0_ref[...] + a1_ref[...]
        t = jnp.dot(h, w1_ref[...], preferred_element_type=jnp.float32)
        t = 0.5 * t * (1.0 + lax.erf(t * 0.7071067811865476))
        y = xb + jnp.dot(t, w2_ref[...], preferred_element_type=jnp.float32)
        mean = jnp.mean(y, axis=1, keepdims=True)
        var = jnp.mean((y - mean) ** 2, axis=1, keepdims=True)
        o_ref[...] = (y - mean) * lax.rsqrt(var + 1e-5) * g_ref[...] + b_ref[...]

    nb = N // BN
    return pl.pallas_call(
        body,
        grid=(nb,),
        in_specs=[
            pl.BlockSpec((BN, D), lambda i: (i, 0)),
            pl.BlockSpec((BN, D), lambda i: (i, 0)),
            pl.BlockSpec((BN, D), lambda i: (i + nb, 0)),
            pl.BlockSpec((D, H), lambda i: (0, 0)),
            pl.BlockSpec((H, D), lambda i: (0, 0)),
            pl.BlockSpec((1, D), lambda i: (0, 0)),
            pl.BlockSpec((1, D), lambda i: (0, 0)),
        ],
        out_specs=pl.BlockSpec((BN, D), lambda i: (i, 0)),
        out_shape=jax.ShapeDtypeStruct((N, D), jnp.float32),
    )(x, agg2, agg2, W1, W2, ln_gamma.reshape(1, D), ln_beta.reshape(1, D))


def kernel(x, edge_index, edge_attr, W1, W2, ln_gamma, ln_beta):
    src = edge_index[0]
    dst = edge_index[1]
    agg2 = _edge_stage(x, src, dst, edge_attr)
    return _dense_stage(x, agg2, W1, W2, ln_gamma, ln_beta)
